# Initial kernel scaffold; baseline (speedup 1.0000x reference)
import jax
import jax.numpy as jnp
from jax.experimental import pallas as pl


def kernel(x, edge_index, hidden_state, p_node_id, encoder_outputs, mask, W1, b1, W2, b2, emb_table, W_attn, b_attn, v_w, W_mlp, b_mlp, W_ih, W_hh, b_ih, b_hh):
    raise NotImplementedError("write your pallas kernel here")


# speedup vs baseline: 7.6415x; 7.6415x over previous
import functools

import jax
import jax.numpy as jnp
from jax import lax
from jax.experimental import pallas as pl
from jax.experimental.pallas import tpu as pltpu
from jax.experimental.pallas import tpu_sc as plsc

N_NODES = 16000
N_EDGES = 256000
B = 16
NPG = 1000
D_FEAT = 256
D = 128

NC = 2
NS = 16
NW = NC * NS
EPW = N_EDGES // NW
CH = 80
NCHUNK = EPW // CH
RPS = N_NODES // NS


@functools.cache
def _make_sc_deg():
    return functools.partial(
        pl.kernel,
        out_type=jax.ShapeDtypeStruct((N_NODES, D), jnp.float32),
        mesh=plsc.VectorSubcoreMesh(core_axis_name="c", subcore_axis_name="s"),
        scratch_types=[
            pltpu.VMEM((CH,), jnp.int32),
            pltpu.VMEM((CH, D), jnp.float32),
            pltpu.VMEM((BR, D), jnp.float32),
            pltpu.VMEM_SHARED((ACC_R, D), jnp.float32),
        ],
    )(_sc_deg_body)


def _sc_deg_body(dst_hbm, ones_hbm, zeros_hbm, out_hbm,
                 idx_d, ones_v, bounce, acc_sh):
    c = lax.axis_index("c")
    s = lax.axis_index("s")
    base = s * EPS
    pltpu.sync_copy(ones_hbm, ones_v)
    pltpu.sync_copy(zeros_hbm, bounce)

    @pl.when(s < 8)
    def _():
        def zstep(k, carry):
            pltpu.sync_copy(bounce, acc_sh.at[pl.ds(s * 1000 + k * BR, BR)])
            return carry
        lax.fori_loop(0, 1000 // BR, zstep, 0)

    @pl.when(s == 8)
    def _():
        pltpu.sync_copy(bounce.at[pl.ds(0, ZR)], acc_sh.at[pl.ds(HN, ZR)])

    plsc.subcore_barrier()

    def step(i, carry):
        pltpu.sync_copy(dst_hbm.at[pl.ds(base + i * CH, CH)], idx_d)
        for j in range(CH // 16):
            sl = pl.ds(j * 16, 16)
            v = idx_d[sl]
            lo = v - c * HN
            ok = jnp.logical_and(lo >= 0, lo < HN)
            idx_d[sl] = jnp.where(ok, lo, HN + jnp.bitwise_and(v, 7))
        pltpu.sync_copy(ones_v, acc_sh.at[idx_d], add=True)
        return carry

    lax.fori_loop(0, NCHUNK_E, step, 0)
    plsc.subcore_barrier()

    @pl.when(s < 8)
    def _():
        def dstep(k, carry):
            pltpu.sync_copy(acc_sh.at[pl.ds(s * 1000 + k * BR, BR)], bounce)
            pltpu.sync_copy(bounce,
                            out_hbm.at[pl.ds(c * HN + s * 1000 + k * BR, BR)])
            return carry
        lax.fori_loop(0, 1000 // BR, dstep, 0)


ZR = 8
BR = 200
HN = N_NODES // NC
ACC_R = HN + 8
EPS = N_EDGES // NS
NCHUNK_E = EPS // CH


@functools.cache
def _make_sc_scat():
    return functools.partial(
        pl.kernel,
        out_type=jax.ShapeDtypeStruct((N_NODES, D), jnp.float32),
        mesh=plsc.VectorSubcoreMesh(core_axis_name="c", subcore_axis_name="s"),
        scratch_types=[
            pltpu.VMEM((CH,), jnp.int32),
            pltpu.VMEM((CH,), jnp.int32),
            pltpu.VMEM((CH, D), jnp.float32),
            pltpu.VMEM((BR, D), jnp.float32),
            pltpu.VMEM_SHARED((ACC_R, D), jnp.float32),
            pltpu.SemaphoreType.DMA,
        ],
    )(_sc_scat_body)


def _sc_scat_body(g_hbm, src_hbm, dst_hbm, zeros_hbm, out_hbm,
                  idx_s, idx_d, rows, bounce, acc_sh, sem):
    c = lax.axis_index("c")
    s = lax.axis_index("s")
    base = s * EPS
    pltpu.sync_copy(zeros_hbm, bounce)

    @pl.when(s < 8)
    def _():
        def zstep(k, carry):
            pltpu.sync_copy(bounce, acc_sh.at[pl.ds(s * 1000 + k * BR, BR)])
            return carry
        lax.fori_loop(0, 1000 // BR, zstep, 0)

    @pl.when(s == 8)
    def _():
        pltpu.sync_copy(bounce.at[pl.ds(0, ZR)], acc_sh.at[pl.ds(HN, ZR)])

    plsc.subcore_barrier()

    def step(i, carry):
        off = base + i * CH
        pltpu.sync_copy(src_hbm.at[pl.ds(off, CH)], idx_s)
        pltpu.sync_copy(dst_hbm.at[pl.ds(off, CH)], idx_d)
        for j in range(CH // 16):
            sl = pl.ds(j * 16, 16)
            v = idx_d[sl]
            lo = v - c * HN
            ok = jnp.logical_and(lo >= 0, lo < HN)
            idx_d[sl] = jnp.where(ok, lo, HN + jnp.bitwise_and(v, 7))
        pltpu.async_copy(g_hbm.at[idx_s], rows, sem).wait()
        pltpu.sync_copy(rows, acc_sh.at[idx_d], add=True)
        return carry

    lax.fori_loop(0, NCHUNK_E, step, 0)
    plsc.subcore_barrier()

    @pl.when(s < 8)
    def _():
        def dstep(k, carry):
            pltpu.sync_copy(acc_sh.at[pl.ds(s * 1000 + k * BR, BR)], bounce)
            pltpu.sync_copy(bounce,
                            out_hbm.at[pl.ds(c * HN + s * 1000 + k * BR, BR)])
            return carry
        lax.fori_loop(0, 1000 // BR, dstep, 0)


@functools.cache
def _make_sc_emb():
    return functools.partial(
        pl.kernel,
        out_type=jax.ShapeDtypeStruct((B, D), jnp.float32),
        mesh=plsc.VectorSubcoreMesh(core_axis_name="c", subcore_axis_name="s"),
        scratch_types=[
            pltpu.VMEM((B,), jnp.int32),
            pltpu.VMEM((B, D), jnp.float32),
            pltpu.SemaphoreType.DMA,
        ],
    )(_sc_emb_body)


def _sc_emb_body(emb_hbm, pid_hbm, out_hbm, pid_v, rows_v, sem):
    c = lax.axis_index("c")
    s = lax.axis_index("s")

    @pl.when(jnp.logical_and(c == 0, s == 0))
    def _():
        pltpu.sync_copy(pid_hbm, pid_v)
        pltpu.async_copy(emb_hbm.at[pid_v], rows_v, sem).wait()
        pltpu.sync_copy(rows_v, out_hbm)


def _tc_mm1_body(x_ref, w1_ref, degp_ref, g1_ref, dinv_ref):
    deg = degp_ref[...] + 1.0
    dinv = lax.rsqrt(deg)
    xw = jnp.dot(x_ref[...], w1_ref[...], preferred_element_type=jnp.float32)
    g1_ref[...] = xw * dinv
    dinv_ref[...] = dinv


def _tc_mm1(x, W1, degp):
    R = 2000
    grid = N_NODES // R
    return pl.pallas_call(
        _tc_mm1_body,
        grid=(grid,),
        in_specs=[
            pl.BlockSpec((R, D_FEAT), lambda i: (i, 0)),
            pl.BlockSpec((D_FEAT, D), lambda i: (0, 0)),
            pl.BlockSpec((R, 1), lambda i: (i, 0)),
        ],
        out_specs=[
            pl.BlockSpec((R, D), lambda i: (i, 0)),
            pl.BlockSpec((R, 1), lambda i: (i, 0)),
        ],
        out_shape=[
            jax.ShapeDtypeStruct((N_NODES, D), jnp.float32),
            jax.ShapeDtypeStruct((N_NODES, 1), jnp.float32),
        ],
    )(x, W1, degp)


def _tc_mm2_body(s1_ref, g1_ref, dinv_ref, b1_ref, w2_ref, b2_ref,
                 g2_ref, u2_ref):
    dinv = dinv_ref[...]
    agg1 = dinv * (s1_ref[...] + g1_ref[...]) + b1_ref[...]
    h1 = jnp.maximum(agg1, 0.0)
    xw2 = jnp.dot(h1, w2_ref[...], preferred_element_type=jnp.float32)
    g2 = xw2 * dinv
    g2_ref[...] = g2
    u2_ref[...] = dinv * g2 + b2_ref[...]


def _tc_mm2(s1, g1, dinv, b1, W2, b2):
    R = 2000
    grid = N_NODES // R
    return pl.pallas_call(
        _tc_mm2_body,
        grid=(grid,),
        in_specs=[
            pl.BlockSpec((R, D), lambda i: (i, 0)),
            pl.BlockSpec((R, D), lambda i: (i, 0)),
            pl.BlockSpec((R, 1), lambda i: (i, 0)),
            pl.BlockSpec((1, D), lambda i: (0, 0)),
            pl.BlockSpec((D, D), lambda i: (0, 0)),
            pl.BlockSpec((1, D), lambda i: (0, 0)),
        ],
        out_specs=[
            pl.BlockSpec((R, D), lambda i: (i, 0)),
            pl.BlockSpec((R, D), lambda i: (i, 0)),
        ],
        out_shape=[
            jax.ShapeDtypeStruct((N_NODES, D), jnp.float32),
            jax.ShapeDtypeStruct((N_NODES, D), jnp.float32),
        ],
    )(s1, g1, dinv, b1, W2, b2)


def _tc_fin_body(s2_ref, u2_ref, dinv_ref, hid_row_ref, wmlp_ref, bmlp_ref,
                 xt_ref, hid_ref, wih_t_ref, whh_t_ref, bih_ref, bhh_ref,
                 logit_ref, hnew_ref):
    i = pl.program_id(0)
    p = dinv_ref[...] * s2_ref[...] + u2_ref[...]
    ph = p + hid_row_ref[0]
    logit_ref[...] = (
        jnp.dot(ph, wmlp_ref[...], preferred_element_type=jnp.float32)
        + bmlp_ref[...]
    )

    @pl.when(i == 0)
    def _():
        x_t = xt_ref[...]
        h_t = hid_ref[...]
        gi = jnp.dot(x_t, wih_t_ref[...], preferred_element_type=jnp.float32) \
            + bih_ref[...]
        gh = jnp.dot(h_t, whh_t_ref[...], preferred_element_type=jnp.float32) \
            + bhh_ref[...]
        r = jax.nn.sigmoid(gi[:, :D] + gh[:, :D])
        z = jax.nn.sigmoid(gi[:, D:2 * D] + gh[:, D:2 * D])
        n_ = jnp.tanh(gi[:, 2 * D:] + r * gh[:, 2 * D:])
        hnew_ref[...] = (1.0 - z) * n_ + z * h_t


def _tc_fin(s2, u2, dinv, hid2d, W_mlp, bmlp2d, x_t, W_ihT, W_hhT,
            bih2d, bhh2d):
    return pl.pallas_call(
        _tc_fin_body,
        grid=(B,),
        in_specs=[
            pl.BlockSpec((NPG, D), lambda i: (i, 0)),
            pl.BlockSpec((NPG, D), lambda i: (i, 0)),
            pl.BlockSpec((NPG, 1), lambda i: (i, 0)),
            pl.BlockSpec((1, 1, D), lambda i: (i, 0, 0)),
            pl.BlockSpec((D, 1), lambda i: (0, 0)),
            pl.BlockSpec((1, 1), lambda i: (0, 0)),
            pl.BlockSpec((B, D), lambda i: (0, 0)),
            pl.BlockSpec((B, D), lambda i: (0, 0)),
            pl.BlockSpec((D, 3 * D), lambda i: (0, 0)),
            pl.BlockSpec((D, 3 * D), lambda i: (0, 0)),
            pl.BlockSpec((1, 3 * D), lambda i: (0, 0)),
            pl.BlockSpec((1, 3 * D), lambda i: (0, 0)),
        ],
        out_specs=[
            pl.BlockSpec((NPG, 1), lambda i: (i, 0)),
            pl.BlockSpec((B, D), lambda i: (0, 0)),
        ],
        out_shape=[
            jax.ShapeDtypeStruct((N_NODES, 1), jnp.float32),
            jax.ShapeDtypeStruct((B, D), jnp.float32),
        ],
    )(s2, u2, dinv, hid2d[:, None, :], W_mlp, bmlp2d, x_t, hid2d,
      W_ihT, W_hhT, bih2d, bhh2d)


def kernel(x, edge_index, hidden_state, p_node_id, encoder_outputs, mask,
           W1, b1, W2, b2, emb_table, W_attn, b_attn, v_w, W_mlp, b_mlp,
           W_ih, W_hh, b_ih, b_hh):
    src = edge_index[0].astype(jnp.int32)
    dst = edge_index[1].astype(jnp.int32)
    pid = p_node_id.astype(jnp.int32)

    ones_w = jnp.ones((CH, D), jnp.float32)
    zeros_z = jnp.zeros((BR, D), jnp.float32)

    sc_deg, sc_scat, sc_emb = _make_sc_deg(), _make_sc_scat(), _make_sc_emb()
    degc = sc_deg(dst, ones_w, zeros_z)
    degp = degc[:, 0:1]
    g1, dinv = _tc_mm1(x, W1, degp)
    s1 = sc_scat(g1, src, dst, zeros_z)
    g2, u2 = _tc_mm2(s1, g1, dinv, b1[None, :], W2, b2[None, :])
    s2 = sc_scat(g2, src, dst, zeros_z)
    x_t = sc_emb(emb_table, pid)

    hid2d = hidden_state[:, 0, :]
    logit_col, h_new = _tc_fin(
        s2, u2, dinv, hid2d, W_mlp, b_mlp[None, :], x_t,
        W_ih.T, W_hh.T, b_ih[None, :], b_hh[None, :])

    logits = logit_col.reshape(B, NPG)
    outputs = h_new[None]
    hidden_out = h_new[None]
    return logits, outputs, hidden_out


# speedup vs baseline: 12.0331x; 1.5747x over previous
import functools

import jax
import jax.numpy as jnp
from jax import lax
from jax.experimental import pallas as pl
from jax.experimental.pallas import tpu as pltpu
from jax.experimental.pallas import tpu_sc as plsc

N_NODES = 16000
N_EDGES = 256000
B = 16
NPG = 1000
D_FEAT = 256
D = 128

NC = 2
NS = 16
NW = NC * NS
EPW = N_EDGES // NW
CH = 80
NCHUNK = EPW // CH
RPS = N_NODES // NS

def _load_remap(dst_hbm, off, idx_ref, c):
    pltpu.sync_copy(dst_hbm.at[pl.ds(off, CH)], idx_ref)
    for j in range(CH // 16):
        sl = pl.ds(j * 16, 16)
        v = idx_ref[sl]
        lo = v - c * HN
        ok = jnp.logical_and(lo >= 0, lo < HN)
        idx_ref[sl] = jnp.where(ok, lo, HN + jnp.bitwise_and(v, 7))


@functools.cache
def _make_sc_deg():
    return functools.partial(
        pl.kernel,
        out_type=jax.ShapeDtypeStruct((N_NODES, D), jnp.float32),
        mesh=plsc.VectorSubcoreMesh(core_axis_name="c", subcore_axis_name="s"),
        scratch_types=[
            pltpu.VMEM((CH,), jnp.int32),
            pltpu.VMEM((CH,), jnp.int32),
            pltpu.VMEM((CH, D), jnp.float32),
            pltpu.VMEM((BR, D), jnp.float32),
            pltpu.VMEM_SHARED((ACC_R, D), jnp.float32),
            pltpu.SemaphoreType.DMA,
            pltpu.SemaphoreType.DMA,
        ],
    )(_sc_deg_body)


def _sc_deg_body(dst_hbm, ones_hbm, zeros_hbm, out_hbm,
                 idx_d, idx_d1, ones_v, bounce, acc_sh, sem0, sem1):
    c = lax.axis_index("c")
    s = lax.axis_index("s")
    base = s * EPS
    pltpu.sync_copy(ones_hbm, ones_v)
    pltpu.sync_copy(zeros_hbm, bounce)

    @pl.when(s < 8)
    def _():
        def zstep(k, carry):
            pltpu.sync_copy(bounce, acc_sh.at[pl.ds(s * 1000 + k * BR, BR)])
            return carry
        lax.fori_loop(0, 1000 // BR, zstep, 0)

    @pl.when(s == 8)
    def _():
        pltpu.sync_copy(bounce.at[pl.ds(0, ZR)], acc_sh.at[pl.ds(HN, ZR)])

    plsc.subcore_barrier()

    def step(i, carry):
        _load_remap(dst_hbm, base + (2 * i) * CH, idx_d, c)
        s0 = pltpu.async_copy(ones_v, acc_sh.at[idx_d], sem0, add=True)
        _load_remap(dst_hbm, base + (2 * i + 1) * CH, idx_d1, c)
        s1 = pltpu.async_copy(ones_v, acc_sh.at[idx_d1], sem1, add=True)
        s0.wait()
        s1.wait()
        return carry

    lax.fori_loop(0, NCHUNK_E // 2, step, 0)
    plsc.subcore_barrier()

    @pl.when(s < 8)
    def _():
        def dstep(k, carry):
            pltpu.sync_copy(acc_sh.at[pl.ds(s * 1000 + k * BR, BR)], bounce)
            pltpu.sync_copy(bounce,
                            out_hbm.at[pl.ds(c * HN + s * 1000 + k * BR, BR)])
            return carry
        lax.fori_loop(0, 1000 // BR, dstep, 0)


ZR = 8
BR = 200
HN = N_NODES // NC
ACC_R = HN + 8
EPS = N_EDGES // NS
NCHUNK_E = EPS // CH


@functools.cache
def _make_sc_scat():
    return functools.partial(
        pl.kernel,
        out_type=jax.ShapeDtypeStruct((N_NODES, D), jnp.float32),
        mesh=plsc.VectorSubcoreMesh(core_axis_name="c", subcore_axis_name="s"),
        scratch_types=[
            pltpu.VMEM((CH,), jnp.int32),
            pltpu.VMEM((CH,), jnp.int32),
            pltpu.VMEM((CH,), jnp.int32),
            pltpu.VMEM((CH,), jnp.int32),
            pltpu.VMEM((CH, D), jnp.float32),
            pltpu.VMEM((CH, D), jnp.float32),
            pltpu.VMEM((BR, D), jnp.float32),
            pltpu.VMEM_SHARED((ACC_R, D), jnp.float32),
            pltpu.SemaphoreType.DMA,
            pltpu.SemaphoreType.DMA,
            pltpu.SemaphoreType.DMA,
            pltpu.SemaphoreType.DMA,
        ],
    )(_sc_scat_body)


def _sc_scat_body(g_hbm, src_hbm, dst_hbm, zeros_hbm, out_hbm,
                  idx_s, idx_s1, idx_d, idx_d1, rows, rows1, bounce, acc_sh,
                  semg0, semg1, sems0, sems1):
    c = lax.axis_index("c")
    s = lax.axis_index("s")
    base = s * EPS
    pltpu.sync_copy(zeros_hbm, bounce)

    @pl.when(s < 8)
    def _():
        def zstep(k, carry):
            pltpu.sync_copy(bounce, acc_sh.at[pl.ds(s * 1000 + k * BR, BR)])
            return carry
        lax.fori_loop(0, 1000 // BR, zstep, 0)

    @pl.when(s == 8)
    def _():
        pltpu.sync_copy(bounce.at[pl.ds(0, ZR)], acc_sh.at[pl.ds(HN, ZR)])

    plsc.subcore_barrier()

    def step(i, carry):
        off0 = base + (2 * i) * CH
        off1 = base + (2 * i + 1) * CH
        pltpu.sync_copy(src_hbm.at[pl.ds(off0, CH)], idx_s)
        g0 = pltpu.async_copy(g_hbm.at[idx_s], rows, semg0)
        pltpu.sync_copy(src_hbm.at[pl.ds(off1, CH)], idx_s1)
        g1 = pltpu.async_copy(g_hbm.at[idx_s1], rows1, semg1)
        _load_remap(dst_hbm, off0, idx_d, c)
        _load_remap(dst_hbm, off1, idx_d1, c)
        g0.wait()
        s0 = pltpu.async_copy(rows, acc_sh.at[idx_d], sems0, add=True)
        g1.wait()
        s1 = pltpu.async_copy(rows1, acc_sh.at[idx_d1], sems1, add=True)
        s0.wait()
        s1.wait()
        return carry

    lax.fori_loop(0, NCHUNK_E // 2, step, 0)
    plsc.subcore_barrier()

    @pl.when(s < 8)
    def _():
        def dstep(k, carry):
            pltpu.sync_copy(acc_sh.at[pl.ds(s * 1000 + k * BR, BR)], bounce)
            pltpu.sync_copy(bounce,
                            out_hbm.at[pl.ds(c * HN + s * 1000 + k * BR, BR)])
            return carry
        lax.fori_loop(0, 1000 // BR, dstep, 0)


@functools.cache
def _make_sc_emb():
    return functools.partial(
        pl.kernel,
        out_type=jax.ShapeDtypeStruct((B, D), jnp.float32),
        mesh=plsc.VectorSubcoreMesh(core_axis_name="c", subcore_axis_name="s"),
        scratch_types=[
            pltpu.VMEM((B,), jnp.int32),
            pltpu.VMEM((B, D), jnp.float32),
            pltpu.SemaphoreType.DMA,
        ],
    )(_sc_emb_body)


def _sc_emb_body(emb_hbm, pid_hbm, out_hbm, pid_v, rows_v, sem):
    c = lax.axis_index("c")
    s = lax.axis_index("s")

    @pl.when(jnp.logical_and(c == 0, s == 0))
    def _():
        pltpu.sync_copy(pid_hbm, pid_v)
        pltpu.async_copy(emb_hbm.at[pid_v], rows_v, sem).wait()
        pltpu.sync_copy(rows_v, out_hbm)


def _tc_mm1_body(x_ref, w1_ref, degp_ref, g1_ref, dinv_ref):
    deg = degp_ref[...] + 1.0
    dinv = lax.rsqrt(deg)
    xw = jnp.dot(x_ref[...], w1_ref[...], preferred_element_type=jnp.float32)
    g1_ref[...] = xw * dinv
    dinv_ref[...] = dinv


def _tc_mm1(x, W1, degp):
    R = 2000
    grid = N_NODES // R
    return pl.pallas_call(
        _tc_mm1_body,
        grid=(grid,),
        in_specs=[
            pl.BlockSpec((R, D_FEAT), lambda i: (i, 0)),
            pl.BlockSpec((D_FEAT, D), lambda i: (0, 0)),
            pl.BlockSpec((R, 1), lambda i: (i, 0)),
        ],
        out_specs=[
            pl.BlockSpec((R, D), lambda i: (i, 0)),
            pl.BlockSpec((R, 1), lambda i: (i, 0)),
        ],
        out_shape=[
            jax.ShapeDtypeStruct((N_NODES, D), jnp.float32),
            jax.ShapeDtypeStruct((N_NODES, 1), jnp.float32),
        ],
    )(x, W1, degp)


def _tc_mm2_body(s1_ref, g1_ref, dinv_ref, b1_ref, w2_ref, b2_ref,
                 g2_ref, u2_ref):
    dinv = dinv_ref[...]
    agg1 = dinv * (s1_ref[...] + g1_ref[...]) + b1_ref[...]
    h1 = jnp.maximum(agg1, 0.0)
    xw2 = jnp.dot(h1, w2_ref[...], preferred_element_type=jnp.float32)
    g2 = xw2 * dinv
    g2_ref[...] = g2
    u2_ref[...] = dinv * g2 + b2_ref[...]


def _tc_mm2(s1, g1, dinv, b1, W2, b2):
    R = 2000
    grid = N_NODES // R
    return pl.pallas_call(
        _tc_mm2_body,
        grid=(grid,),
        in_specs=[
            pl.BlockSpec((R, D), lambda i: (i, 0)),
            pl.BlockSpec((R, D), lambda i: (i, 0)),
            pl.BlockSpec((R, 1), lambda i: (i, 0)),
            pl.BlockSpec((1, D), lambda i: (0, 0)),
            pl.BlockSpec((D, D), lambda i: (0, 0)),
            pl.BlockSpec((1, D), lambda i: (0, 0)),
        ],
        out_specs=[
            pl.BlockSpec((R, D), lambda i: (i, 0)),
            pl.BlockSpec((R, D), lambda i: (i, 0)),
        ],
        out_shape=[
            jax.ShapeDtypeStruct((N_NODES, D), jnp.float32),
            jax.ShapeDtypeStruct((N_NODES, D), jnp.float32),
        ],
    )(s1, g1, dinv, b1, W2, b2)


def _tc_fin_body(s2_ref, u2_ref, dinv_ref, hid_row_ref, wmlp_ref, bmlp_ref,
                 xt_ref, hid_ref, wih_t_ref, whh_t_ref, bih_ref, bhh_ref,
                 logit_ref, hnew_ref):
    i = pl.program_id(0)
    p = dinv_ref[...] * s2_ref[...] + u2_ref[...]
    ph = p + hid_row_ref[0]
    logit_ref[...] = (
        jnp.dot(ph, wmlp_ref[...], preferred_element_type=jnp.float32)
        + bmlp_ref[...]
    )

    @pl.when(i == 0)
    def _():
        x_t = xt_ref[...]
        h_t = hid_ref[...]
        gi = jnp.dot(x_t, wih_t_ref[...], preferred_element_type=jnp.float32) \
            + bih_ref[...]
        gh = jnp.dot(h_t, whh_t_ref[...], preferred_element_type=jnp.float32) \
            + bhh_ref[...]
        r = jax.nn.sigmoid(gi[:, :D] + gh[:, :D])
        z = jax.nn.sigmoid(gi[:, D:2 * D] + gh[:, D:2 * D])
        n_ = jnp.tanh(gi[:, 2 * D:] + r * gh[:, 2 * D:])
        hnew_ref[...] = (1.0 - z) * n_ + z * h_t


def _tc_fin(s2, u2, dinv, hid2d, W_mlp, bmlp2d, x_t, W_ihT, W_hhT,
            bih2d, bhh2d):
    return pl.pallas_call(
        _tc_fin_body,
        grid=(B,),
        in_specs=[
            pl.BlockSpec((NPG, D), lambda i: (i, 0)),
            pl.BlockSpec((NPG, D), lambda i: (i, 0)),
            pl.BlockSpec((NPG, 1), lambda i: (i, 0)),
            pl.BlockSpec((1, 1, D), lambda i: (i, 0, 0)),
            pl.BlockSpec((D, 1), lambda i: (0, 0)),
            pl.BlockSpec((1, 1), lambda i: (0, 0)),
            pl.BlockSpec((B, D), lambda i: (0, 0)),
            pl.BlockSpec((B, D), lambda i: (0, 0)),
            pl.BlockSpec((D, 3 * D), lambda i: (0, 0)),
            pl.BlockSpec((D, 3 * D), lambda i: (0, 0)),
            pl.BlockSpec((1, 3 * D), lambda i: (0, 0)),
            pl.BlockSpec((1, 3 * D), lambda i: (0, 0)),
        ],
        out_specs=[
            pl.BlockSpec((NPG, 1), lambda i: (i, 0)),
            pl.BlockSpec((B, D), lambda i: (0, 0)),
        ],
        out_shape=[
            jax.ShapeDtypeStruct((N_NODES, 1), jnp.float32),
            jax.ShapeDtypeStruct((B, D), jnp.float32),
        ],
    )(s2, u2, dinv, hid2d[:, None, :], W_mlp, bmlp2d, x_t, hid2d,
      W_ihT, W_hhT, bih2d, bhh2d)


def kernel(x, edge_index, hidden_state, p_node_id, encoder_outputs, mask,
           W1, b1, W2, b2, emb_table, W_attn, b_attn, v_w, W_mlp, b_mlp,
           W_ih, W_hh, b_ih, b_hh):
    src = edge_index[0].astype(jnp.int32)
    dst = edge_index[1].astype(jnp.int32)
    pid = p_node_id.astype(jnp.int32)

    ones_w = jnp.ones((CH, D), jnp.float32)
    zeros_z = jnp.zeros((BR, D), jnp.float32)

    sc_deg, sc_scat, sc_emb = _make_sc_deg(), _make_sc_scat(), _make_sc_emb()
    degc = sc_deg(dst, ones_w, zeros_z)
    degp = degc[:, 0:1]
    g1, dinv = _tc_mm1(x, W1, degp)
    s1 = sc_scat(g1, src, dst, zeros_z)
    g2, u2 = _tc_mm2(s1, g1, dinv, b1[None, :], W2, b2[None, :])
    s2 = sc_scat(g2, src, dst, zeros_z)
    x_t = sc_emb(emb_table, pid)

    hid2d = hidden_state[:, 0, :]
    logit_col, h_new = _tc_fin(
        s2, u2, dinv, hid2d, W_mlp, b_mlp[None, :], x_t,
        W_ih.T, W_hh.T, b_ih[None, :], b_hh[None, :])

    logits = logit_col.reshape(B, NPG)
    outputs = h_new[None]
    hidden_out = h_new[None]
    return logits, outputs, hidden_out


# speedup vs baseline: 12.6818x; 1.0539x over previous
import functools

import jax
import jax.numpy as jnp
from jax import lax
from jax.experimental import pallas as pl
from jax.experimental.pallas import tpu as pltpu
from jax.experimental.pallas import tpu_sc as plsc

N_NODES = 16000
N_EDGES = 256000
B = 16
NPG = 1000
D_FEAT = 256
D = 128

NC = 2
NS = 16
NW = NC * NS
EPW = N_EDGES // NW
CH = 80
NCHUNK = EPW // CH
RPS = N_NODES // NS

def _load_remap(dst_hbm, off, idx_ref, c):
    pltpu.sync_copy(dst_hbm.at[pl.ds(off, CH)], idx_ref)
    for j in range(CH // 16):
        sl = pl.ds(j * 16, 16)
        v = idx_ref[sl]
        lo = v - c * HN
        ok = jnp.logical_and(lo >= 0, lo < HN)
        idx_ref[sl] = jnp.where(ok, lo, HN + jnp.bitwise_and(v, 7))


@functools.cache
def _make_sc_deg():
    return functools.partial(
        pl.kernel,
        out_type=jax.ShapeDtypeStruct((N_NODES, D), jnp.float32),
        mesh=plsc.VectorSubcoreMesh(core_axis_name="c", subcore_axis_name="s"),
        scratch_types=[
            pltpu.VMEM((CH,), jnp.int32),
            pltpu.VMEM((CH,), jnp.int32),
            pltpu.VMEM((CH, D), jnp.float32),
            pltpu.VMEM((BR, D), jnp.float32),
            pltpu.VMEM_SHARED((ACC_R, D), jnp.float32),
            pltpu.SemaphoreType.DMA,
            pltpu.SemaphoreType.DMA,
        ],
    )(_sc_deg_body)


def _sc_deg_body(dst_hbm, ones_hbm, zeros_hbm, out_hbm,
                 idx_d, idx_d1, ones_v, bounce, acc_sh, sem0, sem1):
    c = lax.axis_index("c")
    s = lax.axis_index("s")
    base = s * EPS
    pltpu.sync_copy(ones_hbm, ones_v)
    pltpu.sync_copy(zeros_hbm, bounce)

    @pl.when(s < 8)
    def _():
        def zstep(k, carry):
            pltpu.sync_copy(bounce, acc_sh.at[pl.ds(s * 1000 + k * BR, BR)])
            return carry
        lax.fori_loop(0, 1000 // BR, zstep, 0)

    @pl.when(s == 8)
    def _():
        pltpu.sync_copy(bounce.at[pl.ds(0, ZR)], acc_sh.at[pl.ds(HN, ZR)])

    plsc.subcore_barrier()

    def step(i, carry):
        @pl.when(i > 0)
        def _():
            pltpu.make_async_copy(ones_v, acc_sh.at[idx_d], sem0).wait()
        _load_remap(dst_hbm, base + (2 * i) * CH, idx_d, c)
        pltpu.async_copy(ones_v, acc_sh.at[idx_d], sem0, add=True)

        @pl.when(i > 0)
        def _():
            pltpu.make_async_copy(ones_v, acc_sh.at[idx_d1], sem1).wait()
        _load_remap(dst_hbm, base + (2 * i + 1) * CH, idx_d1, c)
        pltpu.async_copy(ones_v, acc_sh.at[idx_d1], sem1, add=True)
        return carry

    lax.fori_loop(0, NCHUNK_E // 2, step, 0)
    pltpu.make_async_copy(ones_v, acc_sh.at[idx_d], sem0).wait()
    pltpu.make_async_copy(ones_v, acc_sh.at[idx_d1], sem1).wait()
    plsc.subcore_barrier()

    @pl.when(s < 8)
    def _():
        def dstep(k, carry):
            pltpu.sync_copy(acc_sh.at[pl.ds(s * 1000 + k * BR, BR)], bounce)
            pltpu.sync_copy(bounce,
                            out_hbm.at[pl.ds(c * HN + s * 1000 + k * BR, BR)])
            return carry
        lax.fori_loop(0, 1000 // BR, dstep, 0)


ZR = 8
BR = 200
HN = N_NODES // NC
ACC_R = HN + 8
EPS = N_EDGES // NS
NCHUNK_E = EPS // CH


@functools.cache
def _make_sc_scat():
    return functools.partial(
        pl.kernel,
        out_type=jax.ShapeDtypeStruct((N_NODES, D), jnp.float32),
        mesh=plsc.VectorSubcoreMesh(core_axis_name="c", subcore_axis_name="s"),
        scratch_types=[
            pltpu.VMEM((CH,), jnp.int32),
            pltpu.VMEM((CH,), jnp.int32),
            pltpu.VMEM((CH,), jnp.int32),
            pltpu.VMEM((CH,), jnp.int32),
            pltpu.VMEM((CH, D), jnp.float32),
            pltpu.VMEM((CH, D), jnp.float32),
            pltpu.VMEM((BR, D), jnp.float32),
            pltpu.VMEM_SHARED((ACC_R, D), jnp.float32),
            pltpu.SemaphoreType.DMA,
            pltpu.SemaphoreType.DMA,
            pltpu.SemaphoreType.DMA,
            pltpu.SemaphoreType.DMA,
        ],
    )(_sc_scat_body)


def _sc_scat_body(g_hbm, src_hbm, dst_hbm, zeros_hbm, out_hbm,
                  idx_s, idx_s1, idx_d, idx_d1, rows, rows1, bounce, acc_sh,
                  semg0, semg1, sems0, sems1):
    c = lax.axis_index("c")
    s = lax.axis_index("s")
    base = s * EPS
    pltpu.sync_copy(zeros_hbm, bounce)

    @pl.when(s < 8)
    def _():
        def zstep(k, carry):
            pltpu.sync_copy(bounce, acc_sh.at[pl.ds(s * 1000 + k * BR, BR)])
            return carry
        lax.fori_loop(0, 1000 // BR, zstep, 0)

    @pl.when(s == 8)
    def _():
        pltpu.sync_copy(bounce.at[pl.ds(0, ZR)], acc_sh.at[pl.ds(HN, ZR)])

    plsc.subcore_barrier()

    def step(i, carry):
        off0 = base + (2 * i) * CH
        off1 = base + (2 * i + 1) * CH
        @pl.when(i > 0)
        def _():
            pltpu.make_async_copy(rows, acc_sh.at[idx_d], sems0).wait()
            pltpu.make_async_copy(rows1, acc_sh.at[idx_d1], sems1).wait()
        pltpu.sync_copy(src_hbm.at[pl.ds(off0, CH)], idx_s)
        g0 = pltpu.async_copy(g_hbm.at[idx_s], rows, semg0)
        pltpu.sync_copy(src_hbm.at[pl.ds(off1, CH)], idx_s1)
        g1 = pltpu.async_copy(g_hbm.at[idx_s1], rows1, semg1)
        _load_remap(dst_hbm, off0, idx_d, c)
        _load_remap(dst_hbm, off1, idx_d1, c)
        g0.wait()
        pltpu.async_copy(rows, acc_sh.at[idx_d], sems0, add=True)
        g1.wait()
        pltpu.async_copy(rows1, acc_sh.at[idx_d1], sems1, add=True)
        return carry

    lax.fori_loop(0, NCHUNK_E // 2, step, 0)
    pltpu.make_async_copy(rows, acc_sh.at[idx_d], sems0).wait()
    pltpu.make_async_copy(rows1, acc_sh.at[idx_d1], sems1).wait()
    plsc.subcore_barrier()

    @pl.when(s < 8)
    def _():
        def dstep(k, carry):
            pltpu.sync_copy(acc_sh.at[pl.ds(s * 1000 + k * BR, BR)], bounce)
            pltpu.sync_copy(bounce,
                            out_hbm.at[pl.ds(c * HN + s * 1000 + k * BR, BR)])
            return carry
        lax.fori_loop(0, 1000 // BR, dstep, 0)


@functools.cache
def _make_sc_emb():
    return functools.partial(
        pl.kernel,
        out_type=jax.ShapeDtypeStruct((B, D), jnp.float32),
        mesh=plsc.VectorSubcoreMesh(core_axis_name="c", subcore_axis_name="s"),
        scratch_types=[
            pltpu.VMEM((B,), jnp.int32),
            pltpu.VMEM((B, D), jnp.float32),
            pltpu.SemaphoreType.DMA,
        ],
    )(_sc_emb_body)


def _sc_emb_body(emb_hbm, pid_hbm, out_hbm, pid_v, rows_v, sem):
    c = lax.axis_index("c")
    s = lax.axis_index("s")

    @pl.when(jnp.logical_and(c == 0, s == 0))
    def _():
        pltpu.sync_copy(pid_hbm, pid_v)
        pltpu.async_copy(emb_hbm.at[pid_v], rows_v, sem).wait()
        pltpu.sync_copy(rows_v, out_hbm)


def _tc_mm1_body(x_ref, w1_ref, degp_ref, g1_ref, dinv_ref):
    deg = degp_ref[...] + 1.0
    dinv = lax.rsqrt(deg)
    xw = jnp.dot(x_ref[...], w1_ref[...], preferred_element_type=jnp.float32)
    g1_ref[...] = xw * dinv
    dinv_ref[...] = dinv


def _tc_mm1(x, W1, degp):
    R = 2000
    grid = N_NODES // R
    return pl.pallas_call(
        _tc_mm1_body,
        grid=(grid,),
        in_specs=[
            pl.BlockSpec((R, D_FEAT), lambda i: (i, 0)),
            pl.BlockSpec((D_FEAT, D), lambda i: (0, 0)),
            pl.BlockSpec((R, 1), lambda i: (i, 0)),
        ],
        out_specs=[
            pl.BlockSpec((R, D), lambda i: (i, 0)),
            pl.BlockSpec((R, 1), lambda i: (i, 0)),
        ],
        out_shape=[
            jax.ShapeDtypeStruct((N_NODES, D), jnp.float32),
            jax.ShapeDtypeStruct((N_NODES, 1), jnp.float32),
        ],
    )(x, W1, degp)


def _tc_mm2_body(s1_ref, g1_ref, dinv_ref, b1_ref, w2_ref, b2_ref,
                 g2_ref, u2_ref):
    dinv = dinv_ref[...]
    agg1 = dinv * (s1_ref[...] + g1_ref[...]) + b1_ref[...]
    h1 = jnp.maximum(agg1, 0.0)
    xw2 = jnp.dot(h1, w2_ref[...], preferred_element_type=jnp.float32)
    g2 = xw2 * dinv
    g2_ref[...] = g2
    u2_ref[...] = dinv * g2 + b2_ref[...]


def _tc_mm2(s1, g1, dinv, b1, W2, b2):
    R = 2000
    grid = N_NODES // R
    return pl.pallas_call(
        _tc_mm2_body,
        grid=(grid,),
        in_specs=[
            pl.BlockSpec((R, D), lambda i: (i, 0)),
            pl.BlockSpec((R, D), lambda i: (i, 0)),
            pl.BlockSpec((R, 1), lambda i: (i, 0)),
            pl.BlockSpec((1, D), lambda i: (0, 0)),
            pl.BlockSpec((D, D), lambda i: (0, 0)),
            pl.BlockSpec((1, D), lambda i: (0, 0)),
        ],
        out_specs=[
            pl.BlockSpec((R, D), lambda i: (i, 0)),
            pl.BlockSpec((R, D), lambda i: (i, 0)),
        ],
        out_shape=[
            jax.ShapeDtypeStruct((N_NODES, D), jnp.float32),
            jax.ShapeDtypeStruct((N_NODES, D), jnp.float32),
        ],
    )(s1, g1, dinv, b1, W2, b2)


def _tc_fin_body(s2_ref, u2_ref, dinv_ref, hid_row_ref, wmlp_ref, bmlp_ref,
                 xt_ref, hid_ref, wih_t_ref, whh_t_ref, bih_ref, bhh_ref,
                 logit_ref, hnew_ref):
    i = pl.program_id(0)
    p = dinv_ref[...] * s2_ref[...] + u2_ref[...]
    ph = p + hid_row_ref[0]
    logit_ref[...] = (
        jnp.dot(ph, wmlp_ref[...], preferred_element_type=jnp.float32)
        + bmlp_ref[...]
    )

    @pl.when(i == 0)
    def _():
        x_t = xt_ref[...]
        h_t = hid_ref[...]
        gi = jnp.dot(x_t, wih_t_ref[...], preferred_element_type=jnp.float32) \
            + bih_ref[...]
        gh = jnp.dot(h_t, whh_t_ref[...], preferred_element_type=jnp.float32) \
            + bhh_ref[...]
        r = jax.nn.sigmoid(gi[:, :D] + gh[:, :D])
        z = jax.nn.sigmoid(gi[:, D:2 * D] + gh[:, D:2 * D])
        n_ = jnp.tanh(gi[:, 2 * D:] + r * gh[:, 2 * D:])
        hnew_ref[...] = (1.0 - z) * n_ + z * h_t


def _tc_fin(s2, u2, dinv, hid2d, W_mlp, bmlp2d, x_t, W_ihT, W_hhT,
            bih2d, bhh2d):
    return pl.pallas_call(
        _tc_fin_body,
        grid=(B,),
        in_specs=[
            pl.BlockSpec((NPG, D), lambda i: (i, 0)),
            pl.BlockSpec((NPG, D), lambda i: (i, 0)),
            pl.BlockSpec((NPG, 1), lambda i: (i, 0)),
            pl.BlockSpec((1, 1, D), lambda i: (i, 0, 0)),
            pl.BlockSpec((D, 1), lambda i: (0, 0)),
            pl.BlockSpec((1, 1), lambda i: (0, 0)),
            pl.BlockSpec((B, D), lambda i: (0, 0)),
            pl.BlockSpec((B, D), lambda i: (0, 0)),
            pl.BlockSpec((D, 3 * D), lambda i: (0, 0)),
            pl.BlockSpec((D, 3 * D), lambda i: (0, 0)),
            pl.BlockSpec((1, 3 * D), lambda i: (0, 0)),
            pl.BlockSpec((1, 3 * D), lambda i: (0, 0)),
        ],
        out_specs=[
            pl.BlockSpec((NPG, 1), lambda i: (i, 0)),
            pl.BlockSpec((B, D), lambda i: (0, 0)),
        ],
        out_shape=[
            jax.ShapeDtypeStruct((N_NODES, 1), jnp.float32),
            jax.ShapeDtypeStruct((B, D), jnp.float32),
        ],
    )(s2, u2, dinv, hid2d[:, None, :], W_mlp, bmlp2d, x_t, hid2d,
      W_ihT, W_hhT, bih2d, bhh2d)


def kernel(x, edge_index, hidden_state, p_node_id, encoder_outputs, mask,
           W1, b1, W2, b2, emb_table, W_attn, b_attn, v_w, W_mlp, b_mlp,
           W_ih, W_hh, b_ih, b_hh):
    src = edge_index[0].astype(jnp.int32)
    dst = edge_index[1].astype(jnp.int32)
    pid = p_node_id.astype(jnp.int32)

    ones_w = jnp.ones((CH, D), jnp.float32)
    zeros_z = jnp.zeros((BR, D), jnp.float32)

    sc_deg, sc_scat, sc_emb = _make_sc_deg(), _make_sc_scat(), _make_sc_emb()
    degc = sc_deg(dst, ones_w, zeros_z)
    degp = degc[:, 0:1]
    g1, dinv = _tc_mm1(x, W1, degp)
    s1 = sc_scat(g1, src, dst, zeros_z)
    g2, u2 = _tc_mm2(s1, g1, dinv, b1[None, :], W2, b2[None, :])
    s2 = sc_scat(g2, src, dst, zeros_z)
    x_t = sc_emb(emb_table, pid)

    hid2d = hidden_state[:, 0, :]
    logit_col, h_new = _tc_fin(
        s2, u2, dinv, hid2d, W_mlp, b_mlp[None, :], x_t,
        W_ih.T, W_hh.T, b_ih[None, :], b_hh[None, :])

    logits = logit_col.reshape(B, NPG)
    outputs = h_new[None]
    hidden_out = h_new[None]
    return logits, outputs, hidden_out
